# baseline (speedup 1.0000x reference)
import jax
import jax.numpy as jnp
from jax.experimental import pallas as pl
from jax.experimental.pallas import tpu as pltpu
from jax.experimental.pallas import tpu_sc as plsc

_SHAPE = (64, 32, 32768)
_N = _SHAPE[0] * _SHAPE[1] * _SHAPE[2]
_LANES = 16
_BLK = 4096


def _sc_apply(x_flat, w_vec, bv_vec, n):
    mesh = plsc.VectorSubcoreMesh(core_axis_name="c", subcore_axis_name="s")

    @pl.kernel(
        out_type=jax.ShapeDtypeStruct((n,), jnp.float32),
        mesh=mesh,
        scratch_types=[
            pltpu.VMEM((_LANES,), jnp.float32),
            pltpu.VMEM((_LANES,), jnp.float32),
        ],
    )
    def sck(w_hbm, bv_hbm, x_hbm, o_hbm, w_vmem, bv_vmem):
        pltpu.sync_copy(w_hbm, w_vmem)
        pltpu.sync_copy(bv_hbm, bv_vmem)
        wreg = w_vmem[...]
        breg = bv_vmem[...]

        def body(in_vmem, out_vmem):
            @pl.loop(0, _BLK, step=_LANES)
            def _(i):
                xv = in_vmem[pl.ds(i, _LANES)]
                y = jnp.where(xv < breg, 0.0, xv) * wreg
                out_vmem[pl.ds(i, _LANES)] = jnp.where(y > 1.0, 1.0, y)

        pltpu.emit_pipeline(
            body,
            grid=(n // _BLK,),
            in_specs=[pl.BlockSpec((_BLK,), lambda i: (i,))],
            out_specs=[pl.BlockSpec((_BLK,), lambda i: (i,))],
            core_axis_name=("c", "s"),
            dimension_semantics=(pltpu.PARALLEL,),
        )(x_hbm, o_hbm)

    return sck(w_vec, bv_vec, x_flat)


def kernel(x, w_inc, w_thr, a, b):
    w = w_inc[0, 0] * a[0]
    bv = w_thr[0, 0] * b[0]
    w_vec = jnp.full((_LANES,), w, jnp.float32)
    bv_vec = jnp.full((_LANES,), bv, jnp.float32)
    out = _sc_apply(x.reshape(_N), w_vec, bv_vec, _N)
    return out.reshape(x.shape)


# speedup vs baseline: 1.7883x; 1.7883x over previous
import jax
import jax.numpy as jnp
from jax.experimental import pallas as pl
from jax.experimental.pallas import tpu as pltpu
from jax.experimental.pallas import tpu_sc as plsc

_SHAPE = (64, 32, 32768)
_N = _SHAPE[0] * _SHAPE[1] * _SHAPE[2]
_LANES = 16
_BLK = 4096


def _sc_apply(x_flat, w_vec, bv_vec, n):
    mesh = plsc.VectorSubcoreMesh(core_axis_name="c", subcore_axis_name="s")

    @pl.kernel(
        out_type=jax.ShapeDtypeStruct((n,), jnp.float32),
        mesh=mesh,
        scratch_types=[
            pltpu.VMEM((_LANES,), jnp.float32),
            pltpu.VMEM((_LANES,), jnp.float32),
        ],
    )
    def sck(w_hbm, bv_hbm, x_hbm, o_hbm, w_vmem, bv_vmem):
        pltpu.sync_copy(w_hbm, w_vmem)
        pltpu.sync_copy(bv_hbm, bv_vmem)
        wreg = w_vmem[...]
        breg = bv_vmem[...]

        def body(in_vmem, out_vmem):
            @plsc.parallel_loop(0, _BLK, step=_LANES, unroll=8)
            def _(i):
                xv = in_vmem[pl.ds(i, _LANES)]
                y = jnp.where(xv < breg, 0.0, xv) * wreg
                out_vmem[pl.ds(i, _LANES)] = jnp.where(y > 1.0, 1.0, y)

        pltpu.emit_pipeline(
            body,
            grid=(n // _BLK,),
            in_specs=[pl.BlockSpec((_BLK,), lambda i: (i,))],
            out_specs=[pl.BlockSpec((_BLK,), lambda i: (i,))],
            core_axis_name=("c", "s"),
            dimension_semantics=(pltpu.PARALLEL,),
        )(x_hbm, o_hbm)

    return sck(w_vec, bv_vec, x_flat)


def kernel(x, w_inc, w_thr, a, b):
    w = w_inc[0, 0] * a[0]
    bv = w_thr[0, 0] * b[0]
    w_vec = jnp.full((_LANES,), w, jnp.float32)
    bv_vec = jnp.full((_LANES,), bv, jnp.float32)
    out = _sc_apply(x.reshape(_N), w_vec, bv_vec, _N)
    return out.reshape(x.shape)


# speedup vs baseline: 2.0506x; 1.1467x over previous
import jax
import jax.numpy as jnp
from jax.experimental import pallas as pl
from jax.experimental.pallas import tpu as pltpu
from jax.experimental.pallas import tpu_sc as plsc

_SHAPE = (64, 32, 32768)
_N = _SHAPE[0] * _SHAPE[1] * _SHAPE[2]
_LANES = 16
_BLK = 8192


def _sc_apply(x_flat, w_vec, bv_vec, n):
    mesh = plsc.VectorSubcoreMesh(core_axis_name="c", subcore_axis_name="s")

    @pl.kernel(
        out_type=jax.ShapeDtypeStruct((n,), jnp.float32),
        mesh=mesh,
        scratch_types=[
            pltpu.VMEM((_LANES,), jnp.float32),
            pltpu.VMEM((_LANES,), jnp.float32),
        ],
    )
    def sck(w_hbm, bv_hbm, x_hbm, o_hbm, w_vmem, bv_vmem):
        pltpu.sync_copy(w_hbm, w_vmem)
        pltpu.sync_copy(bv_hbm, bv_vmem)
        wreg = w_vmem[...]
        breg = bv_vmem[...]

        def body(in_vmem, out_vmem):
            @plsc.parallel_loop(0, _BLK, step=_LANES, unroll=16)
            def _(i):
                xv = in_vmem[pl.ds(i, _LANES)]
                y = jnp.where(xv < breg, 0.0, xv * wreg)
                out_vmem[pl.ds(i, _LANES)] = jnp.minimum(y, 1.0)

        pltpu.emit_pipeline(
            body,
            grid=(n // _BLK,),
            in_specs=[pl.BlockSpec((_BLK,), lambda i: (i,))],
            out_specs=[pl.BlockSpec((_BLK,), lambda i: (i,))],
            core_axis_name=("c", "s"),
            dimension_semantics=(pltpu.PARALLEL,),
        )(x_hbm, o_hbm)

    return sck(w_vec, bv_vec, x_flat)


def kernel(x, w_inc, w_thr, a, b):
    w = w_inc[0, 0] * a[0]
    bv = w_thr[0, 0] * b[0]
    w_vec = jnp.full((_LANES,), w, jnp.float32)
    bv_vec = jnp.full((_LANES,), bv, jnp.float32)
    out = _sc_apply(x.reshape(_N), w_vec, bv_vec, _N)
    return out.reshape(x.shape)


# speedup vs baseline: 7.2274x; 3.5245x over previous
import jax
import jax.numpy as jnp
from jax.experimental import pallas as pl
from jax.experimental.pallas import tpu as pltpu
from jax.experimental.pallas import tpu_sc as plsc

_SHAPE = (64, 32, 32768)
_ROWS = _SHAPE[0] * _SHAPE[1]
_COLS = _SHAPE[2]
_LANES = 16
_BLK = 8192


def _sc_apply(x2, w_vec, bv_vec):
    mesh = plsc.VectorSubcoreMesh(core_axis_name="c", subcore_axis_name="s")

    @pl.kernel(
        out_type=jax.ShapeDtypeStruct((_ROWS, _COLS), jnp.float32),
        mesh=mesh,
        scratch_types=[
            pltpu.VMEM((_LANES,), jnp.float32),
            pltpu.VMEM((_LANES,), jnp.float32),
        ],
    )
    def sck(w_hbm, bv_hbm, x_hbm, o_hbm, w_vmem, bv_vmem):
        pltpu.sync_copy(w_hbm, w_vmem)
        pltpu.sync_copy(bv_hbm, bv_vmem)
        wreg = w_vmem[...]
        breg = bv_vmem[...]

        def body(in_vmem, out_vmem):
            @plsc.parallel_loop(0, _BLK, step=_LANES, unroll=16)
            def _(i):
                xv = in_vmem[pl.ds(i, _LANES)]
                y = jnp.where(xv < breg, 0.0, xv * wreg)
                out_vmem[pl.ds(i, _LANES)] = jnp.minimum(y, 1.0)

        pltpu.emit_pipeline(
            body,
            grid=(_ROWS, _COLS // _BLK),
            in_specs=[pl.BlockSpec((None, _BLK), lambda i, j: (i, j))],
            out_specs=[pl.BlockSpec((None, _BLK), lambda i, j: (i, j))],
            core_axis_name=("c", "s"),
            dimension_semantics=(pltpu.PARALLEL, pltpu.PARALLEL),
        )(x_hbm, o_hbm)

    return sck(w_vec, bv_vec, x2)


def kernel(x, w_inc, w_thr, a, b):
    w = w_inc[0, 0] * a[0]
    bv = w_thr[0, 0] * b[0]
    w_vec = jnp.full((_LANES,), w, jnp.float32)
    bv_vec = jnp.full((_LANES,), bv, jnp.float32)
    out = _sc_apply(x.reshape(_ROWS, _COLS), w_vec, bv_vec)
    return out.reshape(x.shape)
